# baseline (speedup 1.0000x reference)
import functools

import numpy as np

import jax
import jax.numpy as jnp
from jax import lax
from jax.experimental import pallas as pl
from jax.experimental.pallas import tpu as pltpu
from jax.experimental.pallas import tpu_sc as plsc

_D = 128
_L = 16
_NCORES = 2
_NSUB = 16
_NW = _NCORES * _NSUB
_B = 16384
_BPW = _B // _NW
_CH = 64
_NCHUNK = _BPW // _CH
_PCOLS = _CH * _L
_OUT_COLS = 2048
_OUT_ROWS = _B * _L // _OUT_COLS


def _tec_body(ent_hbm, rel_hbm, head_hbm, relidx_hbm, tail_hbm, out_hbm,
              hidx, ridx, tidx, hbuf, rbuf, tbuf, accbuf,
              sem_idx, sem_g0, sem_g1, sem_o0, sem_o1):
    wid = lax.axis_index("s") * _NCORES + lax.axis_index("c")
    base = wid * _BPW

    idx_copies = []
    for c in range(_NCHUNK):
        off = base + c * _CH
        idx_copies.append(
            pltpu.async_copy(head_hbm.at[pl.ds(off, _CH)], hidx.at[c], sem_idx))
        idx_copies.append(
            pltpu.async_copy(relidx_hbm.at[pl.ds(off, _CH)], ridx.at[c], sem_idx))
        idx_copies.append(
            pltpu.async_copy(tail_hbm.at[pl.ds(off, _CH)], tidx.at[c], sem_idx))
    for cp in idx_copies:
        cp.wait()

    gather_sems = (sem_g0, sem_g1)
    out_sems = (sem_o0, sem_o1)
    gathers = [None, None]
    out_copies = [None, None]

    def start_gather(c):
        slot = c % 2
        sem = gather_sems[slot]
        gathers[slot] = (
            pltpu.async_copy(ent_hbm.at[hidx.at[c]], hbuf.at[slot], sem),
            pltpu.async_copy(rel_hbm.at[ridx.at[c]], rbuf.at[slot], sem),
            pltpu.async_copy(ent_hbm.at[tidx.at[c]], tbuf.at[slot], sem),
        )

    start_gather(0)
    for c in range(_NCHUNK):
        slot = c % 2
        if c + 1 < _NCHUNK:
            start_gather(c + 1)
        for cp in gathers[slot]:
            cp.wait()
        if out_copies[slot] is not None:
            out_copies[slot].wait()
        hb, rb, tb, ab = hbuf.at[slot], rbuf.at[slot], tbuf.at[slot], accbuf.at[slot]

        def row_fn(iv, carry):
            for u in range(8):
                i = iv * 8 + u
                acc = jnp.zeros((_L,), jnp.float32)
                for j in range(_D // _L):
                    h = hb[i, pl.ds(j * _L, _L)]
                    r = rb[i, pl.ds(j * _L, _L)]
                    t = tb[i, pl.ds(j * _L, _L)]
                    d = (h - t) + r
                    acc = acc + d * d
                ab[pl.ds(i * _L, _L)] = acc
            return carry

        lax.fori_loop(0, _CH // 8, row_fn, 0)
        out_copies[slot] = pltpu.async_copy(
            ab,
            out_hbm.at[4 * wid + c // 2, pl.ds((c % 2) * _PCOLS, _PCOLS)],
            out_sems[slot])

    for cp in out_copies:
        if cp is not None:
            cp.wait()


def _sc_partial_sums(entity_emb, relation_emb, head, relation, tail):
    mesh = plsc.VectorSubcoreMesh(core_axis_name="c", subcore_axis_name="s")
    run = pl.kernel(
        _tec_body,
        mesh=mesh,
        out_type=jax.ShapeDtypeStruct((_OUT_ROWS, _OUT_COLS), jnp.float32),
        scratch_types=[
            pltpu.VMEM((_NCHUNK, _CH), jnp.int32),
            pltpu.VMEM((_NCHUNK, _CH), jnp.int32),
            pltpu.VMEM((_NCHUNK, _CH), jnp.int32),
            pltpu.VMEM((2, _CH, _D), jnp.float32),
            pltpu.VMEM((2, _CH, _D), jnp.float32),
            pltpu.VMEM((2, _CH, _D), jnp.float32),
            pltpu.VMEM((2, _PCOLS), jnp.float32),
            pltpu.SemaphoreType.DMA,
            pltpu.SemaphoreType.DMA,
            pltpu.SemaphoreType.DMA,
            pltpu.SemaphoreType.DMA,
            pltpu.SemaphoreType.DMA,
        ],
    )
    return run(entity_emb, relation_emb, head, relation, tail)


_FOLD_W_NP = np.repeat(np.eye(_OUT_COLS // _L, dtype=np.float32), _L, axis=0)


def _tc_finish_body(p_ref, w_ref, o_ref):
    x = p_ref[...].astype(jnp.bfloat16)
    s = jnp.dot(x, w_ref[...], preferred_element_type=jnp.float32)
    o_ref[...] = jnp.sqrt(s)


def _tc_finish(partials):
    w = jnp.asarray(_FOLD_W_NP, dtype=jnp.bfloat16)
    out = pl.pallas_call(
        _tc_finish_body,
        out_shape=jax.ShapeDtypeStruct((_OUT_ROWS, _OUT_COLS // _L), jnp.float32),
    )(partials, w)
    return out.reshape(_B)


@jax.jit
def kernel(entity_emb, relation_emb, head, relation, tail):
    partials = _sc_partial_sums(entity_emb, relation_emb, head, relation, tail)
    return _tc_finish(partials)


# speedup vs baseline: 1.0750x; 1.0750x over previous
import functools

import numpy as np

import jax
import jax.numpy as jnp
from jax import lax
from jax.experimental import pallas as pl
from jax.experimental.pallas import tpu as pltpu
from jax.experimental.pallas import tpu_sc as plsc

_D = 128
_L = 16
_NCORES = 2
_NSUB = 16
_NW = _NCORES * _NSUB
_B = 16384
_BPW = _B // _NW
_CH = 64
_NCHUNK = _BPW // _CH
_PCOLS = _CH * _L
_OUT_COLS = 2048
_OUT_ROWS = _B * _L // _OUT_COLS


def _tec_body(ent_hbm, rel_hbm, htidx_hbm, relidx_hbm, out_hbm,
              htidx, ridx, htbuf, rbuf, accbuf,
              sem_idx0, sem_idx, sem_g0, sem_g1, sem_out):
    wid = lax.axis_index("s") * _NCORES + lax.axis_index("c")
    base = wid * _BPW

    first = (
        pltpu.async_copy(htidx_hbm.at[wid * _NCHUNK], htidx.at[0], sem_idx0),
        pltpu.async_copy(relidx_hbm.at[pl.ds(base, _CH)], ridx.at[0], sem_idx0),
    )
    rest = []
    for c in range(1, _NCHUNK):
        rest.append(pltpu.async_copy(
            htidx_hbm.at[wid * _NCHUNK + c], htidx.at[c], sem_idx))
        rest.append(pltpu.async_copy(
            relidx_hbm.at[pl.ds(base + c * _CH, _CH)], ridx.at[c], sem_idx))
    for cp in first:
        cp.wait()

    gather_sems = (sem_g0, sem_g1)
    gathers = [None, None]

    def start_gather(c):
        slot = c % 2
        sem = gather_sems[slot]
        gathers[slot] = (
            pltpu.async_copy(ent_hbm.at[htidx.at[c]], htbuf.at[slot], sem),
            pltpu.async_copy(rel_hbm.at[ridx.at[c]], rbuf.at[slot], sem),
        )

    start_gather(0)
    for cp in rest:
        cp.wait()
    for c in range(_NCHUNK):
        slot = c % 2
        if c + 1 < _NCHUNK:
            start_gather(c + 1)
        for cp in gathers[slot]:
            cp.wait()
        hb, rb = htbuf.at[slot], rbuf.at[slot]
        arow = c // 2
        acol = (c % 2) * _PCOLS

        def row_fn(iv, carry):
            for u in range(4):
                i = iv * 4 + u
                acc = jnp.zeros((_L,), jnp.float32)
                for j in range(_D // _L):
                    h = hb[i, pl.ds(j * _L, _L)]
                    r = rb[i, pl.ds(j * _L, _L)]
                    t = hb[_CH + i, pl.ds(j * _L, _L)]
                    d = (h - t) + r
                    acc = acc + d * d
                accbuf[arow, pl.ds(acol + i * _L, _L)] = acc
            return carry

        lax.fori_loop(0, _CH // 4, row_fn, 0)

    pltpu.sync_copy(accbuf, out_hbm.at[pl.ds(4 * wid, _BPW * _L // _OUT_COLS)])
    del sem_out


def _sc_partial_sums(entity_emb, relation_emb, htidx, relation):
    mesh = plsc.VectorSubcoreMesh(core_axis_name="c", subcore_axis_name="s")
    run = pl.kernel(
        _tec_body,
        mesh=mesh,
        out_type=jax.ShapeDtypeStruct((_OUT_ROWS, _OUT_COLS), jnp.float32),
        scratch_types=[
            pltpu.VMEM((_NCHUNK, 2 * _CH), jnp.int32),
            pltpu.VMEM((_NCHUNK, _CH), jnp.int32),
            pltpu.VMEM((2, 2 * _CH, _D), jnp.float32),
            pltpu.VMEM((2, _CH, _D), jnp.float32),
            pltpu.VMEM((_BPW * _L // _OUT_COLS, _OUT_COLS), jnp.float32),
            pltpu.SemaphoreType.DMA,
            pltpu.SemaphoreType.DMA,
            pltpu.SemaphoreType.DMA,
            pltpu.SemaphoreType.DMA,
            pltpu.SemaphoreType.DMA,
        ],
    )
    return run(entity_emb, relation_emb, htidx, relation)


_FOLD_W_NP = np.repeat(np.eye(_OUT_COLS // _L, dtype=np.float32), _L, axis=0)


def _tc_finish_body(p_ref, w_ref, o_ref):
    x = p_ref[...].astype(jnp.bfloat16)
    s = jnp.dot(x, w_ref[...], preferred_element_type=jnp.float32)
    o_ref[...] = jnp.sqrt(s)


def _tc_finish(partials):
    w = jnp.asarray(_FOLD_W_NP, dtype=jnp.bfloat16)
    out = pl.pallas_call(
        _tc_finish_body,
        out_shape=jax.ShapeDtypeStruct((_OUT_ROWS, _OUT_COLS // _L), jnp.float32),
    )(partials, w)
    return out.reshape(_B)


@jax.jit
def kernel(entity_emb, relation_emb, head, relation, tail):
    htidx = jnp.concatenate(
        [head.reshape(_NW, _NCHUNK, _CH), tail.reshape(_NW, _NCHUNK, _CH)],
        axis=-1).reshape(_NW * _NCHUNK, 2 * _CH)
    partials = _sc_partial_sums(entity_emb, relation_emb, htidx, relation)
    return _tc_finish(partials)


# speedup vs baseline: 1.0918x; 1.0157x over previous
import functools

import numpy as np

import jax
import jax.numpy as jnp
from jax import lax
from jax.experimental import pallas as pl
from jax.experimental.pallas import tpu as pltpu
from jax.experimental.pallas import tpu_sc as plsc

_D = 128
_L = 16
_NCORES = 2
_NSUB = 16
_NW = _NCORES * _NSUB
_B = 16384
_BPW = _B // _NW
_CH = 64
_NCHUNK = _BPW // _CH
_PCOLS = _CH * _L
_OUT_COLS = 2048
_OUT_ROWS = _B * _L // _OUT_COLS


def _tec_body(ent_hbm, rel_hbm, htidx_hbm, relidx_hbm, out_hbm,
              htidx, ridx, htbuf, rbuf, accbuf,
              sem_idx0, sem_idx, sem_g0, sem_g1, sem_out):
    wid = lax.axis_index("s") * _NCORES + lax.axis_index("c")
    base = wid * _BPW

    first = (
        pltpu.async_copy(htidx_hbm.at[wid * _NCHUNK], htidx.at[0], sem_idx0),
        pltpu.async_copy(relidx_hbm.at[pl.ds(base, _CH)], ridx.at[0], sem_idx0),
    )
    rest = []
    for c in range(1, _NCHUNK):
        rest.append(pltpu.async_copy(
            htidx_hbm.at[wid * _NCHUNK + c], htidx.at[c], sem_idx))
        rest.append(pltpu.async_copy(
            relidx_hbm.at[pl.ds(base + c * _CH, _CH)], ridx.at[c], sem_idx))
    for cp in first:
        cp.wait()

    gather_sems = (sem_g0, sem_g1)
    gathers = [None, None]

    def start_gather(c):
        slot = c % 2
        sem = gather_sems[slot]
        gathers[slot] = (
            pltpu.async_copy(ent_hbm.at[htidx.at[c]], htbuf.at[slot], sem),
            pltpu.async_copy(rel_hbm.at[ridx.at[c]], rbuf.at[slot], sem),
        )

    start_gather(0)
    for cp in rest:
        cp.wait()
    for c in range(_NCHUNK):
        slot = c % 2
        if c + 1 < _NCHUNK:
            start_gather(c + 1)
        for cp in gathers[slot]:
            cp.wait()
        hb, rb = htbuf.at[slot], rbuf.at[slot]
        arow = c // 2
        acol = (c % 2) * _PCOLS

        def row_fn(iv, carry):
            for u in range(2):
                i = iv * 2 + u
                acc = jnp.zeros((_L,), jnp.float32)
                for j in range(_D // _L):
                    h = hb[i, pl.ds(j * _L, _L)]
                    r = rb[i, pl.ds(j * _L, _L)]
                    t = hb[_CH + i, pl.ds(j * _L, _L)]
                    d = (h - t) + r
                    acc = acc + d * d
                accbuf[arow, pl.ds(acol + i * _L, _L)] = acc
            return carry

        lax.fori_loop(0, _CH // 2, row_fn, 0)

    pltpu.sync_copy(accbuf, out_hbm.at[pl.ds(4 * wid, _BPW * _L // _OUT_COLS)])
    del sem_out


def _sc_partial_sums(entity_emb, relation_emb, htidx, relation):
    mesh = plsc.VectorSubcoreMesh(core_axis_name="c", subcore_axis_name="s")
    run = pl.kernel(
        _tec_body,
        mesh=mesh,
        out_type=jax.ShapeDtypeStruct((_OUT_ROWS, _OUT_COLS), jnp.float32),
        scratch_types=[
            pltpu.VMEM((_NCHUNK, 2 * _CH), jnp.int32),
            pltpu.VMEM((_NCHUNK, _CH), jnp.int32),
            pltpu.VMEM((2, 2 * _CH, _D), jnp.float32),
            pltpu.VMEM((2, _CH, _D), jnp.float32),
            pltpu.VMEM((_BPW * _L // _OUT_COLS, _OUT_COLS), jnp.float32),
            pltpu.SemaphoreType.DMA,
            pltpu.SemaphoreType.DMA,
            pltpu.SemaphoreType.DMA,
            pltpu.SemaphoreType.DMA,
            pltpu.SemaphoreType.DMA,
        ],
    )
    return run(entity_emb, relation_emb, htidx, relation)


_FOLD_W_NP = np.repeat(np.eye(_OUT_COLS // _L, dtype=np.float32), _L, axis=0)


def _tc_finish_body(p_ref, w_ref, o_ref):
    x = p_ref[...].astype(jnp.bfloat16)
    s = jnp.dot(x, w_ref[...], preferred_element_type=jnp.float32)
    o_ref[...] = jnp.sqrt(s)


def _tc_finish(partials):
    w = jnp.asarray(_FOLD_W_NP, dtype=jnp.bfloat16)
    out = pl.pallas_call(
        _tc_finish_body,
        out_shape=jax.ShapeDtypeStruct((_OUT_ROWS, _OUT_COLS // _L), jnp.float32),
    )(partials, w)
    return out.reshape(_B)


@jax.jit
def kernel(entity_emb, relation_emb, head, relation, tail):
    htidx = jnp.concatenate(
        [head.reshape(_NW, _NCHUNK, _CH), tail.reshape(_NW, _NCHUNK, _CH)],
        axis=-1).reshape(_NW * _NCHUNK, 2 * _CH)
    partials = _sc_partial_sums(entity_emb, relation_emb, htidx, relation)
    return _tc_finish(partials)


# speedup vs baseline: 1.0944x; 1.0023x over previous
import functools

import numpy as np

import jax
import jax.numpy as jnp
from jax import lax
from jax.experimental import pallas as pl
from jax.experimental.pallas import tpu as pltpu
from jax.experimental.pallas import tpu_sc as plsc

_D = 128
_L = 16
_NCORES = 2
_NSUB = 16
_NW = _NCORES * _NSUB
_B = 16384
_BPW = _B // _NW
_CH = 64
_NCHUNK = _BPW // _CH
_PCOLS = _CH * _L
_OUT_COLS = 2048
_OUT_ROWS = _B * _L // _OUT_COLS


def _tec_body(ent_hbm, rel_hbm, htidx_hbm, relidx_hbm, out_hbm,
              htidx, ridx, htbuf, rbuf, accbuf,
              sem_idx0, sem_idx, sem_g0, sem_g1, sem_out):
    wid = lax.axis_index("s") * _NCORES + lax.axis_index("c")
    base = wid * _BPW

    first = (
        pltpu.async_copy(htidx_hbm.at[wid * _NCHUNK], htidx.at[0], sem_idx0),
        pltpu.async_copy(relidx_hbm.at[pl.ds(base, _CH)], ridx.at[0], sem_idx0),
    )
    rest = []
    for c in range(1, _NCHUNK):
        rest.append(pltpu.async_copy(
            htidx_hbm.at[wid * _NCHUNK + c], htidx.at[c], sem_idx))
        rest.append(pltpu.async_copy(
            relidx_hbm.at[pl.ds(base + c * _CH, _CH)], ridx.at[c], sem_idx))
    for cp in first:
        cp.wait()

    gather_sems = (sem_g0, sem_g1)
    gathers = [None, None]

    def start_gather(c):
        slot = c % 2
        sem = gather_sems[slot]
        gathers[slot] = (
            pltpu.async_copy(ent_hbm.at[htidx.at[c]], htbuf.at[slot], sem),
            pltpu.async_copy(rel_hbm.at[ridx.at[c]], rbuf.at[slot], sem),
        )

    start_gather(0)
    for cp in rest:
        cp.wait()
    for c in range(_NCHUNK):
        slot = c % 2
        if c + 1 < _NCHUNK:
            start_gather(c + 1)
        for cp in gathers[slot]:
            cp.wait()
        hb, rb = htbuf.at[slot], rbuf.at[slot]
        arow = c // 2
        acol = (c % 2) * _PCOLS

        def row_fn(i, carry):
            if True:
                acc = jnp.zeros((_L,), jnp.float32)
                for j in range(_D // _L):
                    h = hb[i, pl.ds(j * _L, _L)]
                    r = rb[i, pl.ds(j * _L, _L)]
                    t = hb[_CH + i, pl.ds(j * _L, _L)]
                    d = (h - t) + r
                    acc = acc + d * d
                accbuf[arow, pl.ds(acol + i * _L, _L)] = acc
            return carry

        lax.fori_loop(0, _CH, row_fn, 0)

    pltpu.sync_copy(accbuf, out_hbm.at[pl.ds(4 * wid, _BPW * _L // _OUT_COLS)])
    del sem_out


def _sc_partial_sums(entity_emb, relation_emb, htidx, relation):
    mesh = plsc.VectorSubcoreMesh(core_axis_name="c", subcore_axis_name="s")
    run = pl.kernel(
        _tec_body,
        mesh=mesh,
        out_type=jax.ShapeDtypeStruct((_OUT_ROWS, _OUT_COLS), jnp.float32),
        scratch_types=[
            pltpu.VMEM((_NCHUNK, 2 * _CH), jnp.int32),
            pltpu.VMEM((_NCHUNK, _CH), jnp.int32),
            pltpu.VMEM((2, 2 * _CH, _D), jnp.float32),
            pltpu.VMEM((2, _CH, _D), jnp.float32),
            pltpu.VMEM((_BPW * _L // _OUT_COLS, _OUT_COLS), jnp.float32),
            pltpu.SemaphoreType.DMA,
            pltpu.SemaphoreType.DMA,
            pltpu.SemaphoreType.DMA,
            pltpu.SemaphoreType.DMA,
            pltpu.SemaphoreType.DMA,
        ],
    )
    return run(entity_emb, relation_emb, htidx, relation)


_FOLD_W_NP = np.repeat(np.eye(_OUT_COLS // _L, dtype=np.float32), _L, axis=0)


def _tc_finish_body(p_ref, w_ref, o_ref):
    x = p_ref[...].astype(jnp.bfloat16)
    s = jnp.dot(x, w_ref[...], preferred_element_type=jnp.float32)
    o_ref[...] = jnp.sqrt(s)


def _tc_finish(partials):
    w = jnp.asarray(_FOLD_W_NP, dtype=jnp.bfloat16)
    out = pl.pallas_call(
        _tc_finish_body,
        out_shape=jax.ShapeDtypeStruct((_OUT_ROWS, _OUT_COLS // _L), jnp.float32),
    )(partials, w)
    return out.reshape(_B)


@jax.jit
def kernel(entity_emb, relation_emb, head, relation, tail):
    htidx = jnp.concatenate(
        [head.reshape(_NW, _NCHUNK, _CH), tail.reshape(_NW, _NCHUNK, _CH)],
        axis=-1).reshape(_NW * _NCHUNK, 2 * _CH)
    partials = _sc_partial_sums(entity_emb, relation_emb, htidx, relation)
    return _tc_finish(partials)
